# baseline (speedup 1.0000x reference)
import functools

import jax
import jax.numpy as jnp
import numpy as np
from jax import lax
from jax.experimental import pallas as pl
from jax.experimental.pallas import tpu as pltpu
from jax.experimental.pallas import tpu_sc as plsc

NUM_SAMPLES = 4096
_B, _N, _C = 16, 16384, 64
_S = NUM_SAMPLES
_NBUF = 4
_ROWS = _C // 2

_M32 = np.uint64(0xFFFFFFFF)


def _threefry2x32(k1, k2, x0, x1):
    k1 = np.uint64(k1)
    k2 = np.uint64(k2)
    ks = [k1, k2, (k1 ^ k2 ^ np.uint64(0x1BD11BDA)) & _M32]

    def rounds(x0, x1, rots):
        for r in rots:
            x0 = (x0 + x1) & _M32
            rr = np.uint64(r)
            x1 = x0 ^ (((x1 << rr) | (x1 >> (np.uint64(32) - rr))) & _M32)
        return x0, x1

    rot_a, rot_b = (13, 15, 26, 6), (17, 29, 16, 24)
    x0 = (x0 + ks[0]) & _M32
    x1 = (x1 + ks[1]) & _M32
    for i, rots in enumerate((rot_a, rot_b, rot_a, rot_b, rot_a)):
        x0, x1 = rounds(x0, x1, rots)
        x0 = (x0 + ks[(i + 1) % 3]) & _M32
        x1 = (x1 + ks[(i + 2) % 3] + np.uint64(i + 1)) & _M32
    return x0, x1


def _sample_indices():
    b1, b2 = _threefry2x32(0, 42, np.zeros(2, np.uint64),
                           np.arange(2, dtype=np.uint64))
    n = _B * NUM_SAMPLES
    lb1, lb2 = _threefry2x32(b1[1], b2[1], np.zeros(n, np.uint64),
                             np.arange(n, dtype=np.uint64))
    lower = (lb1 ^ lb2) & _M32
    return (lower % np.uint64(_N)).astype(np.int32).reshape(_B, NUM_SAMPLES)


_IDX = _sample_indices()
_IDXP = _IDX.reshape(2, 8, 32, 128).transpose(0, 2, 1, 3).copy()


def _sc_downsample(p5, x4, idxp):
    mesh = plsc.VectorSubcoreMesh(core_axis_name="c", subcore_axis_name="s")

    @functools.partial(
        pl.kernel,
        mesh=mesh,
        compiler_params=pltpu.CompilerParams(needs_layout_passes=False,
                                             use_tc_tiling_on_sc=False),
        out_type=(
            jax.ShapeDtypeStruct((3, _B // 8, _S // 128, 8, 128),
                                 jnp.float32),
            jax.ShapeDtypeStruct((_B * _C // 8, _S // 128, 8, 128),
                                 jnp.float32),
        ),
        scratch_types=[
            pltpu.VMEM((_NBUF, 128, 128), jnp.float32),
            pltpu.VMEM((_NBUF, 32, 128), jnp.float32),
            pltpu.VMEM((32, 128), jnp.int32),
            pltpu.VMEM((3, 16, 128), jnp.float32),
            pltpu.SemaphoreType.DMA((_NBUF,)),
            pltpu.SemaphoreType.DMA((_NBUF,)),
            pltpu.SemaphoreType.DMA,
        ],
    )
    def k(p_hbm, x_hbm, c_hbm, outp_hbm, outx_hbm,
          xbufs, oxbufs, civ, opbuf,
          insem, outsem, ssem):
        info = plsc.get_sparse_core_info()
        wid = lax.axis_index("s") * info.num_cores + lax.axis_index("c")
        b = wid // 2
        half = wid % 2
        base = b * _C + half * _ROWS
        brg = b // 8
        bri = b % 8

        pltpu.sync_copy(c_hbm.at[brg, :, bri, :], civ)

        for kslot in range(_NBUF):
            r = base + kslot
            pltpu.async_copy(x_hbm.at[r // 8, :, r % 8, :], xbufs.at[kslot],
                             insem.at[kslot])

        def outer(i, carry):
            rr = base + i * _NBUF
            for s in range(_NBUF):
                r = rr + s
                pltpu.make_async_copy(x_hbm.at[r // 8, :, r % 8, :],
                                      xbufs.at[s], insem.at[s]).wait()

            @pl.when(i > 0)
            def _wait_out():
                for s in range(_NBUF):
                    q = rr + s - _NBUF
                    pltpu.make_async_copy(oxbufs.at[s],
                                          outx_hbm.at[q // 8, :, q % 8, :],
                                          outsem.at[s]).wait()

            @plsc.parallel_loop(0, _S // 128, step=1, unroll=2)
            def _chunk(orow):
                for u in range(8):
                    cv = civ[orow, pl.ds(u * 16, 16)]
                    hi = jnp.right_shift(cv, 7)
                    lo = jnp.bitwise_and(cv, 127)
                    for s in range(_NBUF):
                        oxbufs[s, orow, pl.ds(u * 16, 16)] = (
                            plsc.load_gather(xbufs.at[s], [hi, lo]))

            for s in range(_NBUF):
                r = rr + s
                pltpu.async_copy(oxbufs.at[s],
                                 outx_hbm.at[r // 8, :, r % 8, :],
                                 outsem.at[s])

            @pl.when(i < _ROWS // _NBUF - 1)
            def _prefetch():
                for s in range(_NBUF):
                    n = rr + s + _NBUF
                    pltpu.async_copy(x_hbm.at[n // 8, :, n % 8, :],
                                     xbufs.at[s], insem.at[s])

            @pl.when(i == _ROWS // _NBUF - 1)
            def _prefetch_p():
                for d in range(3):
                    pltpu.async_copy(p_hbm.at[d, brg, :, bri, :],
                                     xbufs.at[d], insem.at[d])
            return carry

        lax.fori_loop(0, _ROWS // _NBUF, outer, 0)

        for kslot in range(_NBUF):
            r = base + _ROWS - _NBUF + kslot
            pltpu.make_async_copy(oxbufs.at[kslot],
                                  outx_hbm.at[r // 8, :, r % 8, :],
                                  outsem.at[kslot]).wait()

        for d in range(3):
            pltpu.make_async_copy(p_hbm.at[d, brg, :, bri, :], xbufs.at[d],
                                  insem.at[d]).wait()

        for d in range(3):
            @plsc.parallel_loop(0, 16, step=1, unroll=2)
            def _pchunk(t):
                for u in range(8):
                    cv = civ[half * 16 + t, pl.ds(u * 16, 16)]
                    vals = plsc.load_gather(
                        xbufs.at[d],
                        [jnp.right_shift(cv, 7), jnp.bitwise_and(cv, 127)])
                    opbuf[d, t, pl.ds(u * 16, 16)] = vals

        for d in range(3):
            pltpu.async_copy(
                opbuf.at[d],
                outp_hbm.at[d, brg, pl.ds(half * 16, 16), bri, :], ssem)
        for d in range(3):
            pltpu.make_async_copy(
                opbuf.at[d],
                outp_hbm.at[d, brg, pl.ds(half * 16, 16), bri, :],
                ssem).wait()

    return k(p5, x4, idxp)


def kernel(p, x):
    B, N, _ = p.shape
    C = x.shape[1]
    x4 = x.reshape(B * C // 8, 8, N // 128, 128).transpose(0, 2, 1, 3)
    p5 = p.transpose(2, 0, 1).reshape(3, B // 8, 8, N // 128, 128)
    p5 = p5.transpose(0, 1, 3, 2, 4)
    outp5, outx4 = _sc_downsample(p5, x4, jnp.asarray(_IDXP))
    x_s = outx4.transpose(0, 2, 1, 3).reshape(B, C, NUM_SAMPLES)
    p_s = outp5.transpose(1, 3, 2, 4, 0).reshape(B, NUM_SAMPLES, 3)
    return p_s, x_s


# speedup vs baseline: 1.2099x; 1.2099x over previous
import functools

import jax
import jax.numpy as jnp
import numpy as np
from jax import lax
from jax.experimental import pallas as pl
from jax.experimental.pallas import tpu as pltpu
from jax.experimental.pallas import tpu_sc as plsc

NUM_SAMPLES = 4096
_B, _N, _C = 16, 16384, 64
_S = NUM_SAMPLES
_NBUF = 4
_ROWS = _C // 2

_M32 = np.uint64(0xFFFFFFFF)


def _threefry2x32(k1, k2, x0, x1):
    k1 = np.uint64(k1)
    k2 = np.uint64(k2)
    ks = [k1, k2, (k1 ^ k2 ^ np.uint64(0x1BD11BDA)) & _M32]

    def rounds(x0, x1, rots):
        for r in rots:
            x0 = (x0 + x1) & _M32
            rr = np.uint64(r)
            x1 = x0 ^ (((x1 << rr) | (x1 >> (np.uint64(32) - rr))) & _M32)
        return x0, x1

    rot_a, rot_b = (13, 15, 26, 6), (17, 29, 16, 24)
    x0 = (x0 + ks[0]) & _M32
    x1 = (x1 + ks[1]) & _M32
    for i, rots in enumerate((rot_a, rot_b, rot_a, rot_b, rot_a)):
        x0, x1 = rounds(x0, x1, rots)
        x0 = (x0 + ks[(i + 1) % 3]) & _M32
        x1 = (x1 + ks[(i + 2) % 3] + np.uint64(i + 1)) & _M32
    return x0, x1


def _sample_indices():
    b1, b2 = _threefry2x32(0, 42, np.zeros(2, np.uint64),
                           np.arange(2, dtype=np.uint64))
    n = _B * NUM_SAMPLES
    lb1, lb2 = _threefry2x32(b1[1], b2[1], np.zeros(n, np.uint64),
                             np.arange(n, dtype=np.uint64))
    lower = (lb1 ^ lb2) & _M32
    return (lower % np.uint64(_N)).astype(np.int32).reshape(_B, NUM_SAMPLES)


_IDX = _sample_indices()
_IDXP = _IDX.reshape(2, 8, 32, 128).transpose(0, 2, 1, 3).copy()


def _sc_downsample(p5, x4, idxp):
    mesh = plsc.VectorSubcoreMesh(core_axis_name="c", subcore_axis_name="s")

    @functools.partial(
        pl.kernel,
        mesh=mesh,
        compiler_params=pltpu.CompilerParams(needs_layout_passes=False,
                                             use_tc_tiling_on_sc=False),
        out_type=(
            jax.ShapeDtypeStruct((3, _B // 8, _S // 128, 8, 128),
                                 jnp.float32),
            jax.ShapeDtypeStruct((_B * _C // 8, _S // 128, 8, 128),
                                 jnp.float32),
        ),
        scratch_types=[
            pltpu.VMEM((_NBUF, 128, 128), jnp.float32),
            pltpu.VMEM((_NBUF, 32, 128), jnp.float32),
            pltpu.VMEM((32, 128), jnp.int32),
            pltpu.VMEM((3, 16, 128), jnp.float32),
            pltpu.SemaphoreType.DMA((_NBUF,)),
            pltpu.SemaphoreType.DMA((_NBUF,)),
            pltpu.SemaphoreType.DMA,
        ],
    )
    def k(p_hbm, x_hbm, c_hbm, outp_hbm, outx_hbm,
          xbufs, oxbufs, civ, opbuf,
          insem, outsem, ssem):
        info = plsc.get_sparse_core_info()
        wid = lax.axis_index("s") * info.num_cores + lax.axis_index("c")
        b = wid // 2
        half = wid % 2
        base = b * _C + half * _ROWS
        brg = b // 8
        bri = b % 8

        pltpu.sync_copy(c_hbm.at[brg, :, bri, :], civ)

        for kslot in range(_NBUF):
            r = base + kslot
            pltpu.async_copy(x_hbm.at[r // 8, :, r % 8, :], xbufs.at[kslot],
                             insem.at[kslot])

        def outer(i, carry):
            for pair in range(_NBUF // 2):
                s0, s1 = 2 * pair, 2 * pair + 1
                r0 = base + i * _NBUF + 2 * pair
                r1 = r0 + 1
                pltpu.make_async_copy(x_hbm.at[r0 // 8, :, r0 % 8, :],
                                      xbufs.at[s0], insem.at[s0]).wait()
                pltpu.make_async_copy(x_hbm.at[r1 // 8, :, r1 % 8, :],
                                      xbufs.at[s1], insem.at[s1]).wait()

                @pl.when(i > 0)
                def _wait_out():
                    q0, q1 = r0 - _NBUF, r1 - _NBUF
                    pltpu.make_async_copy(oxbufs.at[s0],
                                          outx_hbm.at[q0 // 8, :, q0 % 8, :],
                                          outsem.at[s0]).wait()
                    pltpu.make_async_copy(oxbufs.at[s1],
                                          outx_hbm.at[q1 // 8, :, q1 % 8, :],
                                          outsem.at[s1]).wait()

                @plsc.parallel_loop(0, _S // 128, step=1, unroll=4)
                def _chunk(orow):
                    for u in range(8):
                        cv = civ[orow, pl.ds(u * 16, 16)]
                        hi = jnp.right_shift(cv, 7)
                        lo = jnp.bitwise_and(cv, 127)
                        oxbufs[s0, orow, pl.ds(u * 16, 16)] = (
                            plsc.load_gather(xbufs.at[s0], [hi, lo]))
                        oxbufs[s1, orow, pl.ds(u * 16, 16)] = (
                            plsc.load_gather(xbufs.at[s1], [hi, lo]))

                pltpu.async_copy(oxbufs.at[s0],
                                 outx_hbm.at[r0 // 8, :, r0 % 8, :],
                                 outsem.at[s0])
                pltpu.async_copy(oxbufs.at[s1],
                                 outx_hbm.at[r1 // 8, :, r1 % 8, :],
                                 outsem.at[s1])

                @pl.when(i < _ROWS // _NBUF - 1)
                def _prefetch():
                    n0, n1 = r0 + _NBUF, r1 + _NBUF
                    pltpu.async_copy(x_hbm.at[n0 // 8, :, n0 % 8, :],
                                     xbufs.at[s0], insem.at[s0])
                    pltpu.async_copy(x_hbm.at[n1 // 8, :, n1 % 8, :],
                                     xbufs.at[s1], insem.at[s1])

                @pl.when(i == _ROWS // _NBUF - 1)
                def _prefetch_p():
                    if pair == 0:
                        pltpu.async_copy(p_hbm.at[0, brg, :, bri, :],
                                         xbufs.at[s0], insem.at[s0])
                        pltpu.async_copy(p_hbm.at[1, brg, :, bri, :],
                                         xbufs.at[s1], insem.at[s1])
                    else:
                        pltpu.async_copy(p_hbm.at[2, brg, :, bri, :],
                                         xbufs.at[s0], insem.at[s0])
            return carry

        lax.fori_loop(0, _ROWS // _NBUF, outer, 0)

        for kslot in range(_NBUF):
            r = base + _ROWS - _NBUF + kslot
            pltpu.make_async_copy(oxbufs.at[kslot],
                                  outx_hbm.at[r // 8, :, r % 8, :],
                                  outsem.at[kslot]).wait()

        for d in range(3):
            pltpu.make_async_copy(p_hbm.at[d, brg, :, bri, :], xbufs.at[d],
                                  insem.at[d]).wait()

        for d in range(3):
            @plsc.parallel_loop(0, 16, step=1, unroll=2)
            def _pchunk(t):
                for u in range(8):
                    cv = civ[half * 16 + t, pl.ds(u * 16, 16)]
                    vals = plsc.load_gather(
                        xbufs.at[d],
                        [jnp.right_shift(cv, 7), jnp.bitwise_and(cv, 127)])
                    opbuf[d, t, pl.ds(u * 16, 16)] = vals

        for d in range(3):
            pltpu.async_copy(
                opbuf.at[d],
                outp_hbm.at[d, brg, pl.ds(half * 16, 16), bri, :], ssem)
        for d in range(3):
            pltpu.make_async_copy(
                opbuf.at[d],
                outp_hbm.at[d, brg, pl.ds(half * 16, 16), bri, :],
                ssem).wait()

    return k(p5, x4, idxp)


def kernel(p, x):
    B, N, _ = p.shape
    C = x.shape[1]
    x4 = x.reshape(B * C // 8, 8, N // 128, 128).transpose(0, 2, 1, 3)
    p5 = p.transpose(2, 0, 1).reshape(3, B // 8, 8, N // 128, 128)
    p5 = p5.transpose(0, 1, 3, 2, 4)
    outp5, outx4 = _sc_downsample(p5, x4, jnp.asarray(_IDXP))
    x_s = outx4.transpose(0, 2, 1, 3).reshape(B, C, NUM_SAMPLES)
    p_s = outp5.transpose(1, 3, 2, 4, 0).reshape(B, NUM_SAMPLES, 3)
    return p_s, x_s
